# Initial kernel scaffold; baseline (speedup 1.0000x reference)
import jax
import jax.numpy as jnp
from jax.experimental import pallas as pl


def kernel(input, edge_index, edge_weight, weight):
    raise NotImplementedError("write your pallas kernel here")


# speedup vs baseline: 3.8164x; 3.8164x over previous
import functools

import jax
import jax.numpy as jnp
from jax import lax
from jax.experimental import pallas as pl
from jax.experimental.pallas import tpu as pltpu
from jax.experimental.pallas import tpu_sc as plsc

NC = 2
NS = 16
NW = NC * NS
LANES = 16
CHUNK = 128


def _matmul_body(x_ref, w_ref, o_ref):
    o_ref[...] = jnp.dot(x_ref[...], w_ref[...],
                         preferred_element_type=jnp.float32)


def _matmul(x, w, block_rows=1000):
    n, d_in = x.shape
    d_out = w.shape[1]
    grid = n // block_rows
    return pl.pallas_call(
        _matmul_body,
        grid=(grid,),
        in_specs=[
            pl.BlockSpec((block_rows, d_in), lambda i: (i, 0)),
            pl.BlockSpec((d_in, d_out), lambda i: (0, 0)),
        ],
        out_specs=pl.BlockSpec((block_rows, d_out), lambda i: (i, 0)),
        out_shape=jax.ShapeDtypeStruct((n, d_out), jnp.float32),
    )(x, w)


def _add_body(a_ref, b_ref, o_ref):
    o_ref[...] = a_ref[...] + b_ref[...]


def _add(a, b, block_rows=1000):
    n, d = a.shape
    grid = n // block_rows
    return pl.pallas_call(
        _add_body,
        grid=(grid,),
        in_specs=[
            pl.BlockSpec((block_rows, d), lambda i: (i, 0)),
            pl.BlockSpec((block_rows, d), lambda i: (i, 0)),
        ],
        out_specs=pl.BlockSpec((block_rows, d), lambda i: (i, 0)),
        out_shape=jax.ShapeDtypeStruct((n, d), jnp.float32),
    )(a, b)


def _make_sc_kernel(n, d, e_per_worker):
    chunks = e_per_worker // CHUNK
    rows_per_sub = n // NS
    mesh = plsc.VectorSubcoreMesh(core_axis_name="c", subcore_axis_name="s")

    @functools.partial(
        pl.kernel,
        out_type=jax.ShapeDtypeStruct((NC, n, d), jnp.float32),
        mesh=mesh,
        scratch_types=[
            pltpu.VMEM_SHARED((n, d), jnp.float32),
            pltpu.VMEM((CHUNK,), jnp.int32),
            pltpu.VMEM((CHUNK,), jnp.int32),
            pltpu.VMEM((CHUNK,), jnp.float32),
            pltpu.VMEM((CHUNK, d), jnp.float32),
            pltpu.SemaphoreType.DMA,
        ],
    )
    def sc_kernel(support_hbm, src_hbm, dst_hbm, w_hbm, zeros_hbm, out_hbm,
                  acc, idx_s, idx_d, wbuf, rows, sem):
        cid = lax.axis_index("c")
        sid = lax.axis_index("s")
        wid = sid * NC + cid
        base = wid * e_per_worker

        pltpu.sync_copy(zeros_hbm, acc.at[pl.ds(sid * rows_per_sub,
                                                rows_per_sub)])
        plsc.subcore_barrier()

        def chunk_body(k, carry):
            off = base + k * CHUNK
            pltpu.sync_copy(src_hbm.at[pl.ds(off, CHUNK)], idx_s)
            pltpu.sync_copy(dst_hbm.at[pl.ds(off, CHUNK)], idx_d)
            pltpu.sync_copy(w_hbm.at[pl.ds(off, CHUNK)], wbuf)
            pltpu.async_copy(support_hbm.at[idx_s], rows, sem).wait()

            def scale_group(g, c2):
                e0 = g * LANES
                w16 = wbuf[pl.ds(e0, LANES)]
                for i in range(LANES):
                    wsp = jnp.full((LANES,), w16[i], jnp.float32)
                    for j in range(d // LANES):
                        sl = pl.ds(j * LANES, LANES)
                        rows[e0 + i, sl] = rows[e0 + i, sl] * wsp
                return c2

            lax.fori_loop(0, CHUNK // LANES, scale_group, 0)

            pltpu.sync_copy(rows, acc.at[idx_d], add=True)
            return carry

        lax.fori_loop(0, chunks, chunk_body, 0)

        plsc.subcore_barrier()
        sl = pl.ds(sid * rows_per_sub, rows_per_sub)
        pltpu.sync_copy(acc.at[sl], out_hbm.at[cid, sl])

    return sc_kernel


def kernel(input, edge_index, edge_weight, weight):
    n, d_in = input.shape
    d_out = weight.shape[1]
    e = edge_weight.shape[0]

    support = _matmul(input, weight)

    e_per_worker = ((e + NW * CHUNK - 1) // (NW * CHUNK)) * CHUNK
    pad = NW * e_per_worker - e
    dst = jnp.pad(edge_index[0], (0, pad))
    src = jnp.pad(edge_index[1], (0, pad))
    w = jnp.pad(edge_weight, (0, pad))

    n_acc = ((n + NS * 8 - 1) // (NS * 8)) * (NS * 8)
    zeros = jnp.zeros((n_acc // NS, d_out), jnp.float32)

    sc = _make_sc_kernel(n_acc, d_out, e_per_worker)
    partial = sc(support, src, dst, w, zeros)
    return _add(partial[0], partial[1], block_rows=n_acc // NS)[:n, :]


# speedup vs baseline: 3.9893x; 1.0453x over previous
import functools

import jax
import jax.numpy as jnp
from jax import lax
from jax.experimental import pallas as pl
from jax.experimental.pallas import tpu as pltpu
from jax.experimental.pallas import tpu_sc as plsc

NC = 2
NS = 16
NW = NC * NS
LANES = 16
CHUNK = 64
SEG = 16


def _matmul_body(x_ref, w_ref, o_ref):
    o_ref[...] = jnp.dot(x_ref[...], w_ref[...],
                         preferred_element_type=jnp.float32)


def _matmul(x, w, block_rows=1000):
    n, d_in = x.shape
    d_out = w.shape[1]
    grid = n // block_rows
    return pl.pallas_call(
        _matmul_body,
        grid=(grid,),
        in_specs=[
            pl.BlockSpec((block_rows, d_in), lambda i: (i, 0)),
            pl.BlockSpec((d_in, d_out), lambda i: (0, 0)),
        ],
        out_specs=pl.BlockSpec((block_rows, d_out), lambda i: (i, 0)),
        out_shape=jax.ShapeDtypeStruct((n, d_out), jnp.float32),
    )(x, w)


def _add_body(a_ref, b_ref, o_ref):
    o_ref[...] = a_ref[...] + b_ref[...]


def _add(a, b, block_rows=1000):
    n, d = a.shape
    grid = n // block_rows
    return pl.pallas_call(
        _add_body,
        grid=(grid,),
        in_specs=[
            pl.BlockSpec((block_rows, d), lambda i: (i, 0)),
            pl.BlockSpec((block_rows, d), lambda i: (i, 0)),
        ],
        out_specs=pl.BlockSpec((block_rows, d), lambda i: (i, 0)),
        out_shape=jax.ShapeDtypeStruct((n, d), jnp.float32),
    )(a, b)


def _make_sc_kernel(n, d, e_per_worker):
    chunks = e_per_worker // CHUNK
    rows_per_sub = n // NS
    mesh = plsc.VectorSubcoreMesh(core_axis_name="c", subcore_axis_name="s")

    nseg = chunks // SEG
    seg_edges = SEG * CHUNK

    @functools.partial(
        pl.kernel,
        out_type=jax.ShapeDtypeStruct((NC, n, d), jnp.float32),
        mesh=mesh,
        scratch_types=[
            pltpu.VMEM_SHARED((n, d), jnp.float32),
            pltpu.VMEM((SEG, CHUNK), jnp.int32),
            pltpu.VMEM((SEG, CHUNK), jnp.int32),
            pltpu.VMEM((seg_edges,), jnp.int32),
            pltpu.VMEM((seg_edges,), jnp.int32),
            pltpu.VMEM((seg_edges,), jnp.float32),
            pltpu.VMEM((seg_edges,), jnp.float32),
            pltpu.VMEM((CHUNK, d), jnp.float32),
            pltpu.VMEM((CHUNK, d), jnp.float32),
            pltpu.VMEM((CHUNK, d), jnp.float32),
            pltpu.VMEM((CHUNK, d), jnp.float32),
            pltpu.SemaphoreType.DMA,
            pltpu.SemaphoreType.DMA,
            pltpu.SemaphoreType.DMA,
            pltpu.SemaphoreType.DMA,
            pltpu.SemaphoreType.DMA,
            pltpu.SemaphoreType.DMA,
            pltpu.SemaphoreType.DMA,
            pltpu.SemaphoreType.DMA,
            pltpu.SemaphoreType.DMA,
            pltpu.SemaphoreType.DMA,
        ],
    )
    def sc_kernel(support_hbm, src_hbm, dst_hbm, w_hbm, zeros_hbm, out_hbm,
                  acc, dseg0, dseg1, seg0, seg1, wseg0, wseg1,
                  rows0, rows1, srows0, srows1,
                  sg0, sg1, ss0, ss1, si0, si1, sw0, sw1, sd0, sd1):
        cid = lax.axis_index("c")
        sid = lax.axis_index("s")
        wid = sid * NC + cid
        dseg = (dseg0, dseg1)
        seg = (seg0, seg1)
        wseg = (wseg0, wseg1)
        rows = (rows0, rows1)
        srows = (srows0, srows1)
        sg = (sg0, sg1)
        ss = (ss0, ss1)
        si = (si0, si1)
        sw = (sw0, sw1)
        sd = (sd0, sd1)

        def stage(s, sb):
            pltpu.async_copy(src_hbm.at[wid, s], seg[sb], si[sb])
            pltpu.async_copy(w_hbm.at[wid, s], wseg[sb], sw[sb])
            pltpu.async_copy(dst_hbm.at[wid, s], dseg[sb], sd[sb])

        def stage_wait(s, sb):
            pltpu.make_async_copy(src_hbm.at[wid, s], seg[sb], si[sb]).wait()
            pltpu.make_async_copy(w_hbm.at[wid, s], wseg[sb], sw[sb]).wait()
            pltpu.make_async_copy(dst_hbm.at[wid, s], dseg[sb], sd[sb]).wait()

        def gather(sb, k_local, b):
            pltpu.async_copy(
                support_hbm.at[seg[sb].at[pl.ds(k_local * CHUNK, CHUNK)]],
                rows[b], sg[b])

        def gather_wait(sb, k_local, b):
            pltpu.make_async_copy(
                support_hbm.at[seg[sb].at[pl.ds(k_local * CHUNK, CHUNK)]],
                rows[b], sg[b]).wait()

        def scatter(sb, k_local, b):
            pltpu.async_copy(srows[b], acc.at[dseg[sb].at[k_local]],
                             ss[b], add=True)

        def scatter_wait(b):
            pltpu.make_async_copy(srows[b], acc.at[dseg[0].at[0]],
                                  ss[b]).wait()

        pltpu.sync_copy(zeros_hbm, acc.at[pl.ds(sid * rows_per_sub,
                                                rows_per_sub)])
        stage(0, 0)
        plsc.subcore_barrier()
        stage_wait(0, 0)
        gather(0, 0, 0)
        gather(0, 1, 1)

        def seg_pair_body(sp, carry):
            for sb in range(2):
                s = 2 * sp + sb

                def pair_body(g, c2):
                    @pl.when(jnp.logical_and(g == 1, s < nseg - 1))
                    def _():
                        stage(s + 1, 1 - sb)

                    for b in range(2):
                        k_local = 2 * g + b
                        k_abs = s * SEG + k_local
                        gather_wait(sb, k_local, b)
                        @pl.when(k_abs >= 2)
                        def _():
                            scatter_wait(b)

                        def scale_group(gg, c3):
                            e0 = gg * LANES
                            w16 = wseg[sb][pl.ds(k_local * CHUNK + e0,
                                                 LANES)]
                            for i in range(LANES):
                                wsp = jnp.full((LANES,), w16[i], jnp.float32)
                                for j in range(d // LANES):
                                    sl = pl.ds(j * LANES, LANES)
                                    srows[b][e0 + i, sl] = (
                                        rows[b][e0 + i, sl] * wsp)
                            return c3

                        lax.fori_loop(0, CHUNK // LANES, scale_group, 0)

                        @pl.when(g < SEG // 2 - 1)
                        def _():
                            gather(sb, k_local + 2, b)

                        @pl.when(jnp.logical_and(g == SEG // 2 - 1,
                                                 s < nseg - 1))
                        def _():
                            if b == 0:
                                stage_wait(s + 1, 1 - sb)
                            gather(1 - sb, b, b)

                        scatter(sb, k_local, b)

                    return c2

                lax.fori_loop(0, SEG // 2, pair_body, 0)
            return carry

        lax.fori_loop(0, nseg // 2, seg_pair_body, 0)

        for b in range(2):
            scatter_wait(b)

        plsc.subcore_barrier()
        sl = pl.ds(sid * rows_per_sub, rows_per_sub)
        pltpu.sync_copy(acc.at[sl], out_hbm.at[cid, sl])

    return sc_kernel


def kernel(input, edge_index, edge_weight, weight):
    n, d_in = input.shape
    d_out = weight.shape[1]
    e = edge_weight.shape[0]

    support = _matmul(input, weight)

    seg_edges = SEG * CHUNK
    e_per_worker = ((e + NW * 2 * seg_edges - 1) //
                    (NW * 2 * seg_edges)) * 2 * seg_edges
    pad = NW * e_per_worker - e
    chunks = e_per_worker // CHUNK
    nseg = chunks // SEG
    dst = jnp.pad(edge_index[0], (0, pad)).reshape(NW, nseg, SEG, CHUNK)
    src = jnp.pad(edge_index[1], (0, pad)).reshape(NW, nseg, seg_edges)
    w = jnp.pad(edge_weight, (0, pad)).reshape(NW, nseg, seg_edges)

    n_acc = ((n + NS * 8 - 1) // (NS * 8)) * (NS * 8)
    zeros = jnp.zeros((n_acc // NS, d_out), jnp.float32)

    sc = _make_sc_kernel(n_acc, d_out, e_per_worker)
    partial = sc(support, src, dst, w, zeros)
    return _add(partial[0], partial[1], block_rows=n_acc // NS)[:n, :]
